# baseline (speedup 1.0000x reference)
import functools

import jax
import jax.numpy as jnp
from jax import lax
from jax.experimental import pallas as pl
from jax.experimental.pallas import tpu as pltpu

_NB = 1024


def _fused_kernel(cnt_inv, nblk, xa_ref, ya_ref, feat_ref, fp_ref, w1a_ref,
                  w1b_ref, b1_ref, g1_ref, bt1_ref, w2_ref, b2_ref, g2_ref,
                  bt2_ref, out_ref, h_ref, s1_ref, q1_ref, s2_ref, q2_ref):
    p = pl.program_id(0)
    b = pl.program_id(1)
    j = pl.program_id(2)
    blk = b * nblk + j

    @pl.when(p == 0)
    def _phase0():
        xa = xa_ref[0]
        ya = ya_ref[0]
        nb = xa.shape[0]
        m = ya.shape[0]
        x2 = jnp.sum(xa[:, 0:3] * xa[:, 0:3], axis=1, keepdims=True)
        d = lax.dot_general(
            xa, ya, (((1,), (1,)), ((), ())),
            preferred_element_type=jnp.float32,
            precision=lax.Precision.HIGHEST)

        big = jnp.float32(jnp.inf)
        chunks = [d[:, k * 256:(k + 1) * 256] for k in range(m // 256)]
        r1 = chunks[0]
        r2 = jnp.full_like(r1, big)
        r3 = jnp.full_like(r1, big)
        for c in chunks[1:]:
            hi1 = jnp.maximum(r1, c)
            r1 = jnp.minimum(r1, c)
            hi2 = jnp.maximum(r2, hi1)
            r2 = jnp.minimum(r2, hi1)
            r3 = jnp.minimum(r3, hi2)
        g = jnp.concatenate([r1, r2, r3], axis=1)
        ms = []
        for k in range(3):
            mk = jnp.min(g, axis=1, keepdims=True)
            ms.append(mk)
            if k < 2:
                g = jnp.where(g == mk, big, g)
        inv1 = lax.rsqrt(jnp.maximum(x2 + ms[0], 1e-12))
        inv2 = lax.rsqrt(jnp.maximum(x2 + ms[1], 1e-12))
        inv3 = lax.rsqrt(jnp.maximum(x2 + ms[2], 1e-12))
        total = inv1 + inv2 + inv3
        scat = jnp.where(d == ms[0], inv1,
                         jnp.where(d == ms[1], inv2,
                                   jnp.where(d == ms[2], inv3, 0.0)))

        interp = lax.dot_general(
            scat, fp_ref[0], (((1,), (1,)), ((), ())),
            preferred_element_type=jnp.float32,
            precision=lax.Precision.DEFAULT) * (1.0 / total)
        h1 = (lax.dot_general(w1a_ref[...], interp,
                              (((1,), (1,)), ((), ())),
                              preferred_element_type=jnp.float32,
                              precision=lax.Precision.DEFAULT)
              + lax.dot_general(w1b_ref[...], feat_ref[0],
                                (((1,), (0,)), ((), ())),
                                preferred_element_type=jnp.float32,
                                precision=lax.Precision.DEFAULT)
              + b1_ref[...])
        h_ref[blk] = h1

        @pl.when((b == 0) & (j == 0))
        def _():
            s1_ref[...] = jnp.zeros_like(s1_ref)
            q1_ref[...] = jnp.zeros_like(q1_ref)

        s1_ref[...] += jnp.sum(h1, axis=1, keepdims=True)
        q1_ref[...] += jnp.sum(h1 * h1, axis=1, keepdims=True)

    @pl.when(p == 1)
    def _phase1():
        mean = s1_ref[...] * cnt_inv
        var = q1_ref[...] * cnt_inv - mean * mean
        scale = g1_ref[...] * lax.rsqrt(var + 1e-5)
        shift = bt1_ref[...] - mean * scale
        r = jnp.maximum(h_ref[blk] * scale + shift, 0.0)
        h2 = lax.dot_general(w2_ref[...], r, (((1,), (0,)), ((), ())),
                             preferred_element_type=jnp.float32,
                             precision=lax.Precision.DEFAULT) + b2_ref[...]
        h_ref[blk] = h2

        @pl.when((b == 0) & (j == 0))
        def _():
            s2_ref[...] = jnp.zeros_like(s2_ref)
            q2_ref[...] = jnp.zeros_like(q2_ref)

        s2_ref[...] += jnp.sum(h2, axis=1, keepdims=True)
        q2_ref[...] += jnp.sum(h2 * h2, axis=1, keepdims=True)

    @pl.when(p == 2)
    def _phase2():
        mean = s2_ref[...] * cnt_inv
        var = q2_ref[...] * cnt_inv - mean * mean
        scale = g2_ref[...] * lax.rsqrt(var + 1e-5)
        shift = bt2_ref[...] - mean * scale
        out_ref[0] = jnp.maximum(h_ref[blk] * scale + shift, 0.0)


def kernel(xyz, xyz_prev, features, features_prev, W1, b1, g1, bt1, W2, b2,
           g2, bt2):
    B, N, _ = xyz.shape
    M = xyz_prev.shape[1]
    C = features.shape[1]
    Cp = features_prev.shape[1]
    d1 = W1.shape[0]
    d2o = W2.shape[0]
    nb = _NB
    nblk = N // nb

    xa = jnp.concatenate(
        [xyz, jnp.ones((B, N, 1), jnp.float32)], axis=2)
    ya = jnp.concatenate(
        [-2.0 * xyz_prev,
         jnp.sum(xyz_prev * xyz_prev, axis=2, keepdims=True)], axis=2)
    w1a = W1[:, :Cp]
    w1b = W1[:, Cp:]
    b1c = b1.reshape(d1, 1)
    g1c = g1.reshape(d1, 1)
    bt1c = bt1.reshape(d1, 1)
    b2c = b2.reshape(d2o, 1)
    g2c = g2.reshape(d2o, 1)
    bt2c = bt2.reshape(d2o, 1)
    cnt_inv = 1.0 / float(B * N)

    zero = lambda p, b, j: (0, 0)
    out = pl.pallas_call(
        functools.partial(_fused_kernel, cnt_inv, nblk),
        grid=(3, B, nblk),
        in_specs=[
            pl.BlockSpec((1, nb, 4),
                         lambda p, b, j: (jnp.where(p == 0, b, 0),
                                          jnp.where(p == 0, j, 0), 0)),
            pl.BlockSpec((1, M, 4),
                         lambda p, b, j: (jnp.where(p == 0, b, 0), 0, 0)),
            pl.BlockSpec((1, C, nb),
                         lambda p, b, j: (jnp.where(p == 0, b, 0), 0,
                                          jnp.where(p == 0, j, 0))),
            pl.BlockSpec((1, Cp, M),
                         lambda p, b, j: (jnp.where(p == 0, b, 0), 0, 0)),
            pl.BlockSpec((d1, Cp), zero),
            pl.BlockSpec((d1, C), zero),
            pl.BlockSpec((d1, 1), zero),
            pl.BlockSpec((d1, 1), zero),
            pl.BlockSpec((d1, 1), zero),
            pl.BlockSpec((d2o, d1), zero),
            pl.BlockSpec((d2o, 1), zero),
            pl.BlockSpec((d2o, 1), zero),
            pl.BlockSpec((d2o, 1), zero),
        ],
        out_specs=pl.BlockSpec(
            (1, d2o, nb),
            lambda p, b, j: (jnp.where(p == 2, b, 0), 0,
                             jnp.where(p == 2, j, 0))),
        out_shape=jax.ShapeDtypeStruct((B, d2o, N), jnp.float32),
        scratch_shapes=[
            pltpu.VMEM((B * nblk, d1, nb), jnp.float32),
            pltpu.VMEM((d1, 1), jnp.float32),
            pltpu.VMEM((d1, 1), jnp.float32),
            pltpu.VMEM((d2o, 1), jnp.float32),
            pltpu.VMEM((d2o, 1), jnp.float32),
        ],
    )(xa, ya, features, features_prev, w1a, w1b, b1c, g1c, bt1c, W2, b2c,
      g2c, bt2c)
    return out


# speedup vs baseline: 1.0599x; 1.0599x over previous
import functools

import jax
import jax.numpy as jnp
from jax import lax
from jax.experimental import pallas as pl
from jax.experimental.pallas import tpu as pltpu

_NB = 1024


def _fused_kernel(cnt_inv, nblk, xa_ref, ya_ref, feat_ref, fp_ref, w1a_ref,
                  w1b_ref, b1_ref, g1_ref, bt1_ref, w2_ref, b2_ref, g2_ref,
                  bt2_ref, out_ref, h_ref, s1_ref, q1_ref, s2_ref, q2_ref):
    p = pl.program_id(0)
    b = pl.program_id(1)
    j = pl.program_id(2)
    blk = b * nblk + j

    @pl.when(p == 0)
    def _phase0():
        xa = xa_ref[0]
        ya = ya_ref[0]
        nb = xa.shape[0]
        m = ya.shape[0]
        x2 = jnp.sum(xa[:, 0:3] * xa[:, 0:3], axis=1, keepdims=True)
        d = lax.dot_general(
            xa, ya, (((1,), (1,)), ((), ())),
            preferred_element_type=jnp.float32,
            precision=lax.Precision.HIGHEST)

        big = jnp.float32(jnp.inf)
        chunks = [d[:, k * 128:(k + 1) * 128] for k in range(m // 128)]
        r1 = chunks[0]
        r2 = jnp.full_like(r1, big)
        r3 = jnp.full_like(r1, big)
        for c in chunks[1:]:
            hi1 = jnp.maximum(r1, c)
            r1 = jnp.minimum(r1, c)
            hi2 = jnp.maximum(r2, hi1)
            r2 = jnp.minimum(r2, hi1)
            r3 = jnp.minimum(r3, hi2)
        g = jnp.concatenate([r1, r2, r3], axis=1)
        ms = []
        for k in range(3):
            mk = jnp.min(g, axis=1, keepdims=True)
            ms.append(mk)
            if k < 2:
                g = jnp.where(g == mk, big, g)
        inv1 = lax.rsqrt(jnp.maximum(x2 + ms[0], 1e-12))
        inv2 = lax.rsqrt(jnp.maximum(x2 + ms[1], 1e-12))
        inv3 = lax.rsqrt(jnp.maximum(x2 + ms[2], 1e-12))
        total = inv1 + inv2 + inv3
        scat = jnp.where(d == ms[0], inv1,
                         jnp.where(d == ms[1], inv2,
                                   jnp.where(d == ms[2], inv3, 0.0)))

        interp = lax.dot_general(
            scat, fp_ref[0], (((1,), (1,)), ((), ())),
            preferred_element_type=jnp.float32,
            precision=lax.Precision.DEFAULT) * (1.0 / total)
        h1 = (lax.dot_general(w1a_ref[...], interp,
                              (((1,), (1,)), ((), ())),
                              preferred_element_type=jnp.float32,
                              precision=lax.Precision.DEFAULT)
              + lax.dot_general(w1b_ref[...], feat_ref[0],
                                (((1,), (0,)), ((), ())),
                                preferred_element_type=jnp.float32,
                                precision=lax.Precision.DEFAULT)
              + b1_ref[...])
        h_ref[blk] = h1

        @pl.when((b == 0) & (j == 0))
        def _():
            s1_ref[...] = jnp.zeros_like(s1_ref)
            q1_ref[...] = jnp.zeros_like(q1_ref)

        s1_ref[...] += jnp.sum(h1, axis=1, keepdims=True)
        q1_ref[...] += jnp.sum(h1 * h1, axis=1, keepdims=True)

    @pl.when(p == 1)
    def _phase1():
        mean = s1_ref[...] * cnt_inv
        var = q1_ref[...] * cnt_inv - mean * mean
        scale = g1_ref[...] * lax.rsqrt(var + 1e-5)
        shift = bt1_ref[...] - mean * scale
        r = jnp.maximum(h_ref[blk] * scale + shift, 0.0)
        h2 = lax.dot_general(w2_ref[...], r, (((1,), (0,)), ((), ())),
                             preferred_element_type=jnp.float32,
                             precision=lax.Precision.DEFAULT) + b2_ref[...]
        h_ref[blk] = h2

        @pl.when((b == 0) & (j == 0))
        def _():
            s2_ref[...] = jnp.zeros_like(s2_ref)
            q2_ref[...] = jnp.zeros_like(q2_ref)

        s2_ref[...] += jnp.sum(h2, axis=1, keepdims=True)
        q2_ref[...] += jnp.sum(h2 * h2, axis=1, keepdims=True)

    @pl.when(p == 2)
    def _phase2():
        mean = s2_ref[...] * cnt_inv
        var = q2_ref[...] * cnt_inv - mean * mean
        scale = g2_ref[...] * lax.rsqrt(var + 1e-5)
        shift = bt2_ref[...] - mean * scale
        out_ref[0] = jnp.maximum(h_ref[blk] * scale + shift, 0.0)


def kernel(xyz, xyz_prev, features, features_prev, W1, b1, g1, bt1, W2, b2,
           g2, bt2):
    B, N, _ = xyz.shape
    M = xyz_prev.shape[1]
    C = features.shape[1]
    Cp = features_prev.shape[1]
    d1 = W1.shape[0]
    d2o = W2.shape[0]
    nb = _NB
    nblk = N // nb

    xa = jnp.concatenate(
        [xyz, jnp.ones((B, N, 1), jnp.float32)], axis=2)
    ya = jnp.concatenate(
        [-2.0 * xyz_prev,
         jnp.sum(xyz_prev * xyz_prev, axis=2, keepdims=True)], axis=2)
    w1a = W1[:, :Cp]
    w1b = W1[:, Cp:]
    b1c = b1.reshape(d1, 1)
    g1c = g1.reshape(d1, 1)
    bt1c = bt1.reshape(d1, 1)
    b2c = b2.reshape(d2o, 1)
    g2c = g2.reshape(d2o, 1)
    bt2c = bt2.reshape(d2o, 1)
    cnt_inv = 1.0 / float(B * N)

    zero = lambda p, b, j: (0, 0)
    out = pl.pallas_call(
        functools.partial(_fused_kernel, cnt_inv, nblk),
        grid=(3, B, nblk),
        in_specs=[
            pl.BlockSpec((1, nb, 4),
                         lambda p, b, j: (jnp.where(p == 0, b, 0),
                                          jnp.where(p == 0, j, 0), 0)),
            pl.BlockSpec((1, M, 4),
                         lambda p, b, j: (jnp.where(p == 0, b, 0), 0, 0)),
            pl.BlockSpec((1, C, nb),
                         lambda p, b, j: (jnp.where(p == 0, b, 0), 0,
                                          jnp.where(p == 0, j, 0))),
            pl.BlockSpec((1, Cp, M),
                         lambda p, b, j: (jnp.where(p == 0, b, 0), 0, 0)),
            pl.BlockSpec((d1, Cp), zero),
            pl.BlockSpec((d1, C), zero),
            pl.BlockSpec((d1, 1), zero),
            pl.BlockSpec((d1, 1), zero),
            pl.BlockSpec((d1, 1), zero),
            pl.BlockSpec((d2o, d1), zero),
            pl.BlockSpec((d2o, 1), zero),
            pl.BlockSpec((d2o, 1), zero),
            pl.BlockSpec((d2o, 1), zero),
        ],
        out_specs=pl.BlockSpec(
            (1, d2o, nb),
            lambda p, b, j: (jnp.where(p == 2, b, 0), 0,
                             jnp.where(p == 2, j, 0))),
        out_shape=jax.ShapeDtypeStruct((B, d2o, N), jnp.float32),
        scratch_shapes=[
            pltpu.VMEM((B * nblk, d1, nb), jnp.float32),
            pltpu.VMEM((d1, 1), jnp.float32),
            pltpu.VMEM((d1, 1), jnp.float32),
            pltpu.VMEM((d2o, 1), jnp.float32),
            pltpu.VMEM((d2o, 1), jnp.float32),
        ],
    )(xa, ya, features, features_prev, w1a, w1b, b1c, g1c, bt1c, W2, b2c,
      g2c, bt2c)
    return out
